# baseline (speedup 1.0000x reference)
import jax
import jax.numpy as jnp
from jax.experimental import pallas as pl
from jax.experimental.pallas import tpu as pltpu

_D_MODEL = 2048
_N_EXPERTS = 64
_D_QUERY = 128
_N_HEADS = 2
_D_INT = 512
_N_SUB = 8

_TILE = 512
_N_TOK = 2048
_N_TILES = _N_TOK // _TILE
_QDIM = _N_HEADS * _D_QUERY
_WCHUNK = _D_MODEL // _N_TILES


def _gelu(v):
    return 0.5 * v * (1.0 + jax.lax.erf(v * (2.0 ** -0.5)))


def _max_mask(s, iota):
    m = jnp.max(s, axis=1, keepdims=True)
    am = jnp.min(jnp.where(s == m, iota, s.shape[1]), axis=1, keepdims=True)
    return m, iota == am


def _fused_kernel(x_ref, wq_ref, lat_ref, w1_ref, w2u_ref, wv_ref, bq_ref,
                  gam_ref, bet_ref, skc_ref, out_ref,
                  x_s, q_s, ha_s, g_s, hv_s, stats_s):
    i = pl.program_id(0)

    @pl.when(i == 0)
    def _prep():
        ha_s[...] = _gelu(jnp.dot(lat_ref[...], w1_ref[...],
                                  preferred_element_type=jnp.float32))

    @pl.when(i < _N_TILES)
    def _phase_a():
        xt = x_ref[...]
        x_s[pl.ds(i * _TILE, _TILE), :] = xt
        q = jax.lax.dot_general(xt, wq_ref[...], (((1,), (1,)), ((), ())),
                                preferred_element_type=jnp.float32) + bq_ref[...]
        q_s[pl.ds(i * _TILE, _TILE), :] = q
        ha = ha_s[...]
        g_s[pl.ds(i * _WCHUNK, _WCHUNK), :] = jax.lax.dot_general(
            w2u_ref[...], ha, (((0,), (1,)), ((), ())),
            preferred_element_type=jnp.float32)
        hv_s[:, pl.ds(i * _WCHUNK, _WCHUNK)] = jnp.dot(
            ha, wv_ref[...], preferred_element_type=jnp.float32)
        part = jnp.concatenate(
            [jnp.sum(q, axis=0, keepdims=True),
             jnp.sum(q * q, axis=0, keepdims=True),
             jnp.zeros((6, _QDIM), jnp.float32)], axis=0)

        @pl.when(i == 0)
        def _():
            stats_s[...] = part

        @pl.when(i != 0)
        def _():
            stats_s[...] += part

    @pl.when(i >= _N_TILES)
    def _phase_b():
        j = i - _N_TILES
        stats = stats_s[...]
        mean = stats[0:1, :] * (1.0 / _N_TOK)
        var = stats[1:2, :] * (1.0 / _N_TOK) - mean * mean
        rstd = jax.lax.rsqrt(var + 1e-5)
        qn = ((q_s[pl.ds(j * _TILE, _TILE), :] - mean)
              * (rstd * gam_ref[...]) + bet_ref[...])
        logits = jax.lax.dot_general(
            x_s[pl.ds(j * _TILE, _TILE), :], g_s[...],
            (((1,), (0,)), ((), ())),
            preferred_element_type=jnp.float32)
        glog = _gelu(logits)
        iota64 = jax.lax.broadcasted_iota(jnp.int32, (_TILE, _N_EXPERTS), 1)
        w = jnp.zeros((_TILE, _N_EXPERTS), jnp.float32)
        for h in range(_N_HEADS):
            qh = qn[:, h * _D_QUERY:(h + 1) * _D_QUERY]
            comb = jax.lax.dot_general(qh, skc_ref[...],
                                       (((1,), (1,)), ((), ())),
                                       preferred_element_type=jnp.float32)
            m0, mask0 = _max_mask(comb, iota64)
            m1, mask1 = _max_mask(jnp.where(mask0, -jnp.inf, comb), iota64)
            fs0 = 1.0 / (1.0 + jnp.exp(m1 - m0))
            w = w + glog * (jnp.where(mask0, fs0, 0.0)
                            + jnp.where(mask1, 1.0 - fs0, 0.0))
        out_ref[...] = jax.lax.dot_general(
            w, hv_s[...], (((1,), (0,)), ((), ())),
            preferred_element_type=jnp.float32) * (1.0 / _N_HEADS)


def kernel(x, latents, W1, W2, Wq, bq, gamma, beta, sk1, sk2):
    B, S, D = x.shape
    n_tok = B * S
    xf = x.reshape(n_tok, D)
    WqR = Wq.reshape(_QDIM, D)
    W2u = W2[:, :D]
    Wv = W2[:, D:]
    bqr = bq.reshape(1, -1)
    gam = gamma.reshape(1, -1)
    bet = beta.reshape(1, -1)
    skcomb = jnp.concatenate(
        [jnp.repeat(sk1, _N_SUB, axis=0), jnp.tile(sk2, (_N_SUB, 1))], axis=1)
    f32 = jnp.float32

    nt = _N_TILES
    out = pl.pallas_call(
        _fused_kernel,
        grid=(2 * nt,),
        in_specs=[
            pl.BlockSpec((_TILE, D), lambda i: (jnp.minimum(i, nt - 1), 0)),
            pl.BlockSpec((_QDIM, D), lambda i: (0, 0)),
            pl.BlockSpec((_N_EXPERTS, _D_QUERY), lambda i: (0, 0)),
            pl.BlockSpec((_D_QUERY, _D_INT), lambda i: (0, 0)),
            pl.BlockSpec((_D_INT, _WCHUNK),
                         lambda i: (0, jnp.minimum(i, nt - 1))),
            pl.BlockSpec((_D_INT, _WCHUNK),
                         lambda i: (0, jnp.minimum(i, nt - 1))),
            pl.BlockSpec((1, _QDIM), lambda i: (0, 0)),
            pl.BlockSpec((1, _QDIM), lambda i: (0, 0)),
            pl.BlockSpec((1, _QDIM), lambda i: (0, 0)),
            pl.BlockSpec((_N_EXPERTS, _D_QUERY), lambda i: (0, 0)),
        ],
        out_specs=pl.BlockSpec(
            (_TILE, D), lambda i: (jnp.maximum(i - nt, 0), 0)),
        out_shape=jax.ShapeDtypeStruct((n_tok, D), f32),
        scratch_shapes=[
            pltpu.VMEM((_N_TOK, _D_MODEL), f32),
            pltpu.VMEM((_N_TOK, _QDIM), f32),
            pltpu.VMEM((_N_EXPERTS, _D_INT), f32),
            pltpu.VMEM((_D_MODEL, _N_EXPERTS), f32),
            pltpu.VMEM((_N_EXPERTS, _D_MODEL), f32),
            pltpu.VMEM((8, _QDIM), f32),
        ],
    )(xf, WqR, latents, W1, W2u, Wv, bqr, gam, bet, skcomb)

    return out.reshape(B, S, D)


# speedup vs baseline: 1.2135x; 1.2135x over previous
import jax
import jax.numpy as jnp
from jax.experimental import pallas as pl
from jax.experimental.pallas import tpu as pltpu

_D_MODEL = 2048
_N_EXPERTS = 64
_D_QUERY = 128
_N_HEADS = 2
_D_INT = 512
_N_SUB = 8

_TILE = 512
_N_TOK = 2048
_N_TILES = _N_TOK // _TILE
_QDIM = _N_HEADS * _D_QUERY
_WCHUNK = _D_MODEL // _N_TILES


def _gelu(v):
    return 0.5 * v * (1.0 + jax.lax.erf(v * (2.0 ** -0.5)))


def _max_mask(s, iota):
    m = jnp.max(s, axis=1, keepdims=True)
    am = jnp.min(jnp.where(s == m, iota, s.shape[1]), axis=1, keepdims=True)
    return m, iota == am


def _fused_kernel(x_ref, wq_ref, lat_ref, w1_ref, w2u_ref, wv_ref, bq_ref,
                  gam_ref, bet_ref, skc_ref, out_ref,
                  q_s, l_s, ha_s, g_s, hv_s, stats_s):
    i = pl.program_id(0)

    @pl.when(i == 0)
    def _prep():
        ha_s[...] = _gelu(jnp.dot(lat_ref[...], w1_ref[...],
                                  preferred_element_type=jnp.float32))
        g_s[...] = jax.lax.dot_general(
            w2u_ref[...], ha_s[...], (((0,), (1,)), ((), ())),
            preferred_element_type=jnp.float32)

    @pl.when(i < _N_TILES)
    def _phase_a():
        xt = x_ref[...]
        q = jax.lax.dot_general(xt, wq_ref[...], (((1,), (1,)), ((), ())),
                                preferred_element_type=jnp.float32) + bq_ref[...]
        q_s[pl.ds(i * _TILE, _TILE), :] = q
        l_s[pl.ds(i * _TILE, _TILE), :] = jax.lax.dot_general(
            xt, g_s[...], (((1,), (0,)), ((), ())),
            preferred_element_type=jnp.float32)
        hv_s[:, pl.ds(i * _WCHUNK, _WCHUNK)] = jnp.dot(
            ha_s[...], wv_ref[...], preferred_element_type=jnp.float32)
        part = jnp.concatenate(
            [jnp.sum(q, axis=0, keepdims=True),
             jnp.sum(q * q, axis=0, keepdims=True),
             jnp.zeros((6, _QDIM), jnp.float32)], axis=0)

        @pl.when(i == 0)
        def _():
            stats_s[...] = part

        @pl.when(i != 0)
        def _():
            stats_s[...] += part

    @pl.when(i >= _N_TILES)
    def _phase_b():
        j = i - _N_TILES
        stats = stats_s[...]
        mean = stats[0:1, :] * (1.0 / _N_TOK)
        var = stats[1:2, :] * (1.0 / _N_TOK) - mean * mean
        rstd = jax.lax.rsqrt(var + 1e-5)
        qn = ((q_s[pl.ds(j * _TILE, _TILE), :] - mean)
              * (rstd * gam_ref[...]) + bet_ref[...])
        glog = _gelu(l_s[pl.ds(j * _TILE, _TILE), :])
        iota64 = jax.lax.broadcasted_iota(jnp.int32, (_TILE, _N_EXPERTS), 1)
        w = jnp.zeros((_TILE, _N_EXPERTS), jnp.float32)
        for h in range(_N_HEADS):
            qh = qn[:, h * _D_QUERY:(h + 1) * _D_QUERY]
            comb = jax.lax.dot_general(qh, skc_ref[...],
                                       (((1,), (1,)), ((), ())),
                                       preferred_element_type=jnp.float32)
            m0, mask0 = _max_mask(comb, iota64)
            m1, mask1 = _max_mask(jnp.where(mask0, -jnp.inf, comb), iota64)
            fs0 = 1.0 / (1.0 + jnp.exp(m1 - m0))
            w = w + glog * (jnp.where(mask0, fs0, 0.0)
                            + jnp.where(mask1, 1.0 - fs0, 0.0))
        out_ref[...] = jax.lax.dot_general(
            w, hv_s[...], (((1,), (0,)), ((), ())),
            preferred_element_type=jnp.float32) * (1.0 / _N_HEADS)


def kernel(x, latents, W1, W2, Wq, bq, gamma, beta, sk1, sk2):
    B, S, D = x.shape
    n_tok = B * S
    xf = x.reshape(n_tok, D)
    WqR = Wq.reshape(_QDIM, D)
    W2u = W2[:, :D]
    Wv = W2[:, D:]
    bqr = bq.reshape(1, -1)
    gam = gamma.reshape(1, -1)
    bet = beta.reshape(1, -1)
    skcomb = jnp.concatenate(
        [jnp.repeat(sk1, _N_SUB, axis=0), jnp.tile(sk2, (_N_SUB, 1))], axis=1)
    f32 = jnp.float32

    nt = _N_TILES
    out = pl.pallas_call(
        _fused_kernel,
        grid=(2 * nt,),
        in_specs=[
            pl.BlockSpec((_TILE, D), lambda i: (jnp.minimum(i, nt - 1), 0)),
            pl.BlockSpec((_QDIM, D), lambda i: (0, 0)),
            pl.BlockSpec((_N_EXPERTS, _D_QUERY), lambda i: (0, 0)),
            pl.BlockSpec((_D_QUERY, _D_INT), lambda i: (0, 0)),
            pl.BlockSpec((_D_INT, _D_MODEL), lambda i: (0, 0)),
            pl.BlockSpec((_D_INT, _WCHUNK),
                         lambda i: (0, jnp.minimum(i, nt - 1))),
            pl.BlockSpec((1, _QDIM), lambda i: (0, 0)),
            pl.BlockSpec((1, _QDIM), lambda i: (0, 0)),
            pl.BlockSpec((1, _QDIM), lambda i: (0, 0)),
            pl.BlockSpec((_N_EXPERTS, _D_QUERY), lambda i: (0, 0)),
        ],
        out_specs=pl.BlockSpec(
            (_TILE, D), lambda i: (jnp.maximum(i - nt, 0), 0)),
        out_shape=jax.ShapeDtypeStruct((n_tok, D), f32),
        scratch_shapes=[
            pltpu.VMEM((_N_TOK, _QDIM), f32),
            pltpu.VMEM((_N_TOK, _N_EXPERTS), f32),
            pltpu.VMEM((_N_EXPERTS, _D_INT), f32),
            pltpu.VMEM((_D_MODEL, _N_EXPERTS), f32),
            pltpu.VMEM((_N_EXPERTS, _D_MODEL), f32),
            pltpu.VMEM((8, _QDIM), f32),
        ],
    )(xf, WqR, latents, W1, W2u, Wv, bqr, gam, bet, skcomb)

    return out.reshape(B, S, D)


# speedup vs baseline: 1.6903x; 1.3930x over previous
import jax
import jax.numpy as jnp
from jax.experimental import pallas as pl
from jax.experimental.pallas import tpu as pltpu

_D_MODEL = 2048
_N_EXPERTS = 64
_D_QUERY = 128
_N_HEADS = 2
_D_INT = 512
_N_SUB = 8

_TILE = 512
_N_TOK = 2048
_N_TILES = _N_TOK // _TILE
_QDIM = _N_HEADS * _D_QUERY
_WCHUNK = _D_MODEL // _N_TILES


def _gelu(v):
    return 0.5 * v * (1.0 + jax.lax.erf(v * (2.0 ** -0.5)))


def _max_mask(s, iota):
    m = jnp.max(s, axis=1, keepdims=True)
    am = jnp.min(jnp.where(s == m, iota, s.shape[1]), axis=1, keepdims=True)
    return m, iota == am


def _row_pair(ref):
    return jnp.concatenate([ref[0:1, :], ref[1:2, :]], axis=1)


def _fused_kernel(x_ref, wq_ref, lat_ref, w1_ref, w2u_ref, wv_ref, bq_ref,
                  gam_ref, bet_ref, sk1_ref, sk2_ref, out_ref,
                  q_s, l_s, ha_s, g_s, hv_s, skc_s, stats_s):
    i = pl.program_id(0)

    @pl.when(i == 0)
    def _prep():
        ha_s[...] = _gelu(jnp.dot(lat_ref[...], w1_ref[...],
                                  preferred_element_type=jnp.float32))
        g_s[...] = jax.lax.dot_general(
            w2u_ref[...], ha_s[...], (((0,), (1,)), ((), ())),
            preferred_element_type=jnp.float32)
        r = jax.lax.broadcasted_iota(jnp.int32, (_N_EXPERTS, _N_SUB), 0)
        c = jax.lax.broadcasted_iota(jnp.int32, (_N_EXPERTS, _N_SUB), 1)
        e1 = (r // _N_SUB == c).astype(jnp.float32)
        e2 = (r % _N_SUB == c).astype(jnp.float32)
        skc_s[:, : _D_QUERY // 2] = jnp.dot(
            e1, sk1_ref[...], preferred_element_type=jnp.float32)
        skc_s[:, _D_QUERY // 2:] = jnp.dot(
            e2, sk2_ref[...], preferred_element_type=jnp.float32)

    @pl.when(i < _N_TILES)
    def _phase_a():
        xt = x_ref[...]
        q = jax.lax.dot_general(xt, wq_ref[...], (((1,), (1,)), ((), ())),
                                preferred_element_type=jnp.float32)
        q = q + _row_pair(bq_ref)
        q_s[pl.ds(i * _TILE, _TILE), :] = q
        l_s[pl.ds(i * _TILE, _TILE), :] = jax.lax.dot_general(
            xt, g_s[...], (((1,), (0,)), ((), ())),
            preferred_element_type=jnp.float32)
        hv_s[:, pl.ds(i * _WCHUNK, _WCHUNK)] = jnp.dot(
            ha_s[...], wv_ref[...], preferred_element_type=jnp.float32)
        part = jnp.concatenate(
            [jnp.sum(q, axis=0, keepdims=True),
             jnp.sum(q * q, axis=0, keepdims=True),
             jnp.zeros((6, _QDIM), jnp.float32)], axis=0)

        @pl.when(i == 0)
        def _():
            stats_s[...] = part

        @pl.when(i != 0)
        def _():
            stats_s[...] += part

    @pl.when(i >= _N_TILES)
    def _phase_b():
        j = i - _N_TILES
        stats = stats_s[...]
        mean = stats[0:1, :] * (1.0 / _N_TOK)
        var = stats[1:2, :] * (1.0 / _N_TOK) - mean * mean
        rstd = jax.lax.rsqrt(var + 1e-5)
        qn = ((q_s[pl.ds(j * _TILE, _TILE), :] - mean)
              * (rstd * _row_pair(gam_ref)) + _row_pair(bet_ref))
        glog = _gelu(l_s[pl.ds(j * _TILE, _TILE), :])
        iota64 = jax.lax.broadcasted_iota(jnp.int32, (_TILE, _N_EXPERTS), 1)
        w = jnp.zeros((_TILE, _N_EXPERTS), jnp.float32)
        for h in range(_N_HEADS):
            qh = qn[:, h * _D_QUERY:(h + 1) * _D_QUERY]
            comb = jax.lax.dot_general(qh, skc_s[...],
                                       (((1,), (1,)), ((), ())),
                                       preferred_element_type=jnp.float32)
            m0, mask0 = _max_mask(comb, iota64)
            m1, mask1 = _max_mask(jnp.where(mask0, -jnp.inf, comb), iota64)
            fs0 = 1.0 / (1.0 + jnp.exp(m1 - m0))
            w = w + glog * (jnp.where(mask0, fs0, 0.0)
                            + jnp.where(mask1, 1.0 - fs0, 0.0))
        out_ref[...] = jax.lax.dot_general(
            w, hv_s[...], (((1,), (0,)), ((), ())),
            preferred_element_type=jnp.float32) * (1.0 / _N_HEADS)


def kernel(x, latents, W1, W2, Wq, bq, gamma, beta, sk1, sk2):
    B, S, D = x.shape
    n_tok = B * S
    xf = x.reshape(n_tok, D)
    WqR = Wq.reshape(_QDIM, D)
    f32 = jnp.float32

    nt = _N_TILES
    out = pl.pallas_call(
        _fused_kernel,
        grid=(2 * nt,),
        in_specs=[
            pl.BlockSpec((_TILE, D), lambda i: (jnp.minimum(i, nt - 1), 0)),
            pl.BlockSpec((_QDIM, D), lambda i: (0, 0)),
            pl.BlockSpec((_N_EXPERTS, _D_QUERY), lambda i: (0, 0)),
            pl.BlockSpec((_D_QUERY, _D_INT), lambda i: (0, 0)),
            pl.BlockSpec((_D_INT, _D_MODEL), lambda i: (0, 0)),
            pl.BlockSpec((_D_INT, _WCHUNK),
                         lambda i: (0, nt + jnp.minimum(i, nt - 1))),
            pl.BlockSpec((2, _D_QUERY), lambda i: (0, 0)),
            pl.BlockSpec((2, _D_QUERY), lambda i: (0, 0)),
            pl.BlockSpec((2, _D_QUERY), lambda i: (0, 0)),
            pl.BlockSpec((_N_SUB, _D_QUERY // 2), lambda i: (0, 0)),
            pl.BlockSpec((_N_SUB, _D_QUERY // 2), lambda i: (0, 0)),
        ],
        out_specs=pl.BlockSpec(
            (_TILE, D), lambda i: (jnp.maximum(i - nt, 0), 0)),
        out_shape=jax.ShapeDtypeStruct((n_tok, D), f32),
        scratch_shapes=[
            pltpu.VMEM((_N_TOK, _QDIM), f32),
            pltpu.VMEM((_N_TOK, _N_EXPERTS), f32),
            pltpu.VMEM((_N_EXPERTS, _D_INT), f32),
            pltpu.VMEM((_D_MODEL, _N_EXPERTS), f32),
            pltpu.VMEM((_N_EXPERTS, _D_MODEL), f32),
            pltpu.VMEM((_N_EXPERTS, _D_QUERY), f32),
            pltpu.VMEM((8, _QDIM), f32),
        ],
    )(xf, WqR, latents, W1, W2, W2, bq, gamma, beta, sk1, sk2)

    return out.reshape(B, S, D)
